# baseline (speedup 1.0000x reference)
import functools

import jax
import jax.numpy as jnp
from jax import lax
from jax.experimental import pallas as pl
from jax.experimental.pallas import tpu as pltpu
from jax.experimental.pallas import tpu_sc as plsc

_NC = 2
_NS = 16
_NW = _NC * _NS
_CHUNK = 128

_EPS = 1e-5


def _make_gather(batch, dim, table_len):
    b_per_w = batch // _NW
    n_chunks = b_per_w // _CHUNK
    assert b_per_w * _NW == batch and n_chunks * _CHUNK == b_per_w

    mesh = plsc.VectorSubcoreMesh(core_axis_name="c", subcore_axis_name="s")

    @functools.partial(
        pl.kernel,
        mesh=mesh,
        compiler_params=pltpu.CompilerParams(use_tc_tiling_on_sc=False),
        out_type=jax.ShapeDtypeStruct((batch, dim), jnp.float32),
        scratch_types=[
            pltpu.VMEM((n_chunks, _CHUNK), jnp.int32),
            pltpu.VMEM((b_per_w, dim), jnp.float32),
            pltpu.SemaphoreType.DMA,
        ],
    )
    def gather_k(idx_hbm, table_hbm, out_hbm, idx_v, rows_v, sem):
        wid = lax.axis_index("s") * _NC + lax.axis_index("c")
        base = wid * b_per_w
        pltpu.sync_copy(idx_hbm.at[pl.ds(wid * n_chunks, n_chunks)], idx_v)
        copies = []
        for j in range(n_chunks):
            copies.append(
                pltpu.async_copy(
                    table_hbm.at[idx_v.at[j]],
                    rows_v.at[pl.ds(j * _CHUNK, _CHUNK)],
                    sem,
                )
            )
        for c in copies:
            c.wait()
        pltpu.sync_copy(rows_v, out_hbm.at[pl.ds(base, b_per_w)])

    return gather_k


def _layernorm(x):
    mu = jnp.mean(x, axis=-1, keepdims=True)
    var = jnp.mean((x - mu) ** 2, axis=-1, keepdims=True)
    return (x - mu) / jnp.sqrt(var + _EPS)


def _gelu(x):
    return x * 0.5 * (1.0 + lax.erf(x * (2.0**-0.5)))


def _mlp_body(e_ref, w1_ref, b1_ref, w2_ref, b2_ref, w3_ref, b3_ref, o_ref):
    h = jnp.dot(e_ref[...], w1_ref[...], preferred_element_type=jnp.float32)
    h = _gelu(_layernorm(h + b1_ref[...]))
    h = jnp.dot(h, w2_ref[...], preferred_element_type=jnp.float32)
    h = _gelu(_layernorm(h + b2_ref[...]))
    h = jnp.dot(h, w3_ref[...], preferred_element_type=jnp.float32)
    o_ref[...] = _gelu(h + b3_ref[...])


def _mlp(e, W1, b1, W2, b2, W3, b3, block=2048):
    batch = e.shape[0]
    grid = batch // block
    full = lambda shape: pl.BlockSpec(shape, lambda i: (0, 0))
    return pl.pallas_call(
        _mlp_body,
        grid=(grid,),
        in_specs=[
            pl.BlockSpec((block, e.shape[1]), lambda i: (i, 0)),
            full(W1.shape),
            full(b1.shape),
            full(W2.shape),
            full(b2.shape),
            full(W3.shape),
            full(b3.shape),
        ],
        out_specs=pl.BlockSpec((block, W3.shape[1]), lambda i: (i, 0)),
        out_shape=jax.ShapeDtypeStruct((batch, W3.shape[1]), jnp.float32),
    )(e, W1, b1, W2, b2, W3, b3)


def kernel(movie_ids, table, W1, b1, W2, b2, W3, b3):
    batch = movie_ids.shape[0]
    gather = _make_gather(batch, table.shape[1], table.shape[0])
    idx2d = movie_ids.astype(jnp.int32).reshape(-1, _CHUNK)
    e = gather(idx2d, table)
    return _mlp(
        e,
        W1,
        b1.reshape(1, -1),
        W2,
        b2.reshape(1, -1),
        W3,
        b3.reshape(1, -1),
    )


# speedup vs baseline: 1.0355x; 1.0355x over previous
import functools

import jax
import jax.numpy as jnp
from jax import lax
from jax.experimental import pallas as pl
from jax.experimental.pallas import tpu as pltpu
from jax.experimental.pallas import tpu_sc as plsc

_NC = 2
_NS = 16
_NW = _NC * _NS

_EPS = 1e-5


def _make_gather(batch, dim):
    b_per_w = batch // _NW
    assert b_per_w * _NW == batch

    mesh = plsc.VectorSubcoreMesh(core_axis_name="c", subcore_axis_name="s")

    @functools.partial(
        pl.kernel,
        mesh=mesh,
        compiler_params=pltpu.CompilerParams(needs_layout_passes=False),
        out_type=jax.ShapeDtypeStruct((batch, dim), jnp.float32),
        scratch_types=[
            pltpu.VMEM((b_per_w,), jnp.int32),
            pltpu.SemaphoreType.DMA,
        ],
    )
    def gather_k(idx_hbm, table_hbm, out_hbm, idx_v, sem):
        wid = lax.axis_index("s") * _NC + lax.axis_index("c")
        base = wid * b_per_w
        pltpu.sync_copy(idx_hbm.at[pl.ds(base, b_per_w)], idx_v)
        lanes = lax.iota(jnp.int32, 16)

        def fire(c, _):
            vec = idx_v[pl.ds(c * 16, 16)]

            def one(l, _):
                row = jnp.sum(jnp.where(lanes == l, vec, 0))
                pltpu.make_async_copy(
                    table_hbm.at[row], out_hbm.at[base + c * 16 + l], sem
                ).start()
                return 0

            lax.fori_loop(0, 16, one, 0)
            return 0

        lax.fori_loop(0, b_per_w // 16, fire, 0)

        def drain(i, _):
            pltpu.make_async_copy(
                table_hbm.at[0], out_hbm.at[base], sem
            ).wait()
            return 0

        lax.fori_loop(0, b_per_w, drain, 0)

    return gather_k


def _layernorm(x):
    mu = jnp.mean(x, axis=-1, keepdims=True)
    var = jnp.mean((x - mu) ** 2, axis=-1, keepdims=True)
    return (x - mu) / jnp.sqrt(var + _EPS)


def _gelu(x):
    return x * 0.5 * (1.0 + lax.erf(x * (2.0**-0.5)))


def _mlp_body(e_ref, w1_ref, b1_ref, w2_ref, b2_ref, w3_ref, b3_ref, o_ref):
    h = jnp.dot(e_ref[...], w1_ref[...], preferred_element_type=jnp.float32)
    h = _gelu(_layernorm(h + b1_ref[...]))
    h = jnp.dot(h, w2_ref[...], preferred_element_type=jnp.float32)
    h = _gelu(_layernorm(h + b2_ref[...]))
    h = jnp.dot(h, w3_ref[...], preferred_element_type=jnp.float32)
    o_ref[...] = _gelu(h + b3_ref[...])


def _mlp(e, W1, b1, W2, b2, W3, b3, block=2048):
    batch = e.shape[0]
    grid = batch // block
    full = lambda shape: pl.BlockSpec(shape, lambda i: (0, 0))
    return pl.pallas_call(
        _mlp_body,
        grid=(grid,),
        in_specs=[
            pl.BlockSpec((block, e.shape[1]), lambda i: (i, 0)),
            full(W1.shape),
            full(b1.shape),
            full(W2.shape),
            full(b2.shape),
            full(W3.shape),
            full(b3.shape),
        ],
        out_specs=pl.BlockSpec((block, W3.shape[1]), lambda i: (i, 0)),
        out_shape=jax.ShapeDtypeStruct((batch, W3.shape[1]), jnp.float32),
    )(e, W1, b1, W2, b2, W3, b3)


def kernel(movie_ids, table, W1, b1, W2, b2, W3, b3):
    batch = movie_ids.shape[0]
    gather = _make_gather(batch, table.shape[1])
    e = gather(movie_ids.astype(jnp.int32), table)
    return _mlp(
        e,
        W1,
        b1.reshape(1, -1),
        W2,
        b2.reshape(1, -1),
        W3,
        b3.reshape(1, -1),
    )


# speedup vs baseline: 1.7004x; 1.6421x over previous
import functools

import jax
import jax.numpy as jnp
from jax import lax
from jax.experimental import pallas as pl
from jax.experimental.pallas import tpu as pltpu
from jax.experimental.pallas import tpu_sc as plsc

_NC = 2
_NS = 16
_NW = _NC * _NS

_EPS = 1e-5


def _make_gather(batch, dim):
    b_per_w = batch // _NW
    assert b_per_w * _NW == batch

    mesh = plsc.VectorSubcoreMesh(core_axis_name="c", subcore_axis_name="s")

    @functools.partial(
        pl.kernel,
        mesh=mesh,
        compiler_params=pltpu.CompilerParams(needs_layout_passes=False),
        out_type=jax.ShapeDtypeStruct((batch, dim), jnp.float32),
        scratch_types=[
            pltpu.VMEM((b_per_w,), jnp.int32),
            pltpu.VMEM((b_per_w, dim), jnp.float32),
            pltpu.SemaphoreType.DMA,
        ],
    )
    def gather_k(idx_hbm, table_hbm, out_hbm, idx_v, rows_v, sem):
        wid = lax.axis_index("s") * _NC + lax.axis_index("c")
        base = wid * b_per_w
        pltpu.sync_copy(idx_hbm.at[pl.ds(base, b_per_w)], idx_v)
        lanes = lax.iota(jnp.int32, 16)

        def fire(c, _):
            vec = idx_v[pl.ds(c * 16, 16)]
            for l in range(16):
                row = jnp.sum(jnp.where(lanes == l, vec, 0))
                pltpu.make_async_copy(
                    table_hbm.at[row], rows_v.at[c * 16 + l], sem
                ).start()
            return 0

        lax.fori_loop(0, b_per_w // 16, fire, 0)

        def drain(i, _):
            pltpu.make_async_copy(
                table_hbm.at[0], rows_v.at[0], sem
            ).wait()
            return 0

        lax.fori_loop(0, b_per_w, drain, 0)
        pltpu.sync_copy(rows_v, out_hbm.at[pl.ds(base, b_per_w)])

    return gather_k


def _layernorm(x):
    mu = jnp.mean(x, axis=-1, keepdims=True)
    var = jnp.mean((x - mu) ** 2, axis=-1, keepdims=True)
    return (x - mu) / jnp.sqrt(var + _EPS)


def _gelu(x):
    return x * 0.5 * (1.0 + lax.erf(x * (2.0**-0.5)))


def _mlp_body(e_ref, w1_ref, b1_ref, w2_ref, b2_ref, w3_ref, b3_ref, o_ref):
    h = jnp.dot(e_ref[...], w1_ref[...], preferred_element_type=jnp.float32)
    h = _gelu(_layernorm(h + b1_ref[...]))
    h = jnp.dot(h, w2_ref[...], preferred_element_type=jnp.float32)
    h = _gelu(_layernorm(h + b2_ref[...]))
    h = jnp.dot(h, w3_ref[...], preferred_element_type=jnp.float32)
    o_ref[...] = _gelu(h + b3_ref[...])


def _mlp(e, W1, b1, W2, b2, W3, b3, block=2048):
    batch = e.shape[0]
    grid = batch // block
    full = lambda shape: pl.BlockSpec(shape, lambda i: (0, 0))
    return pl.pallas_call(
        _mlp_body,
        grid=(grid,),
        in_specs=[
            pl.BlockSpec((block, e.shape[1]), lambda i: (i, 0)),
            full(W1.shape),
            full(b1.shape),
            full(W2.shape),
            full(b2.shape),
            full(W3.shape),
            full(b3.shape),
        ],
        out_specs=pl.BlockSpec((block, W3.shape[1]), lambda i: (i, 0)),
        out_shape=jax.ShapeDtypeStruct((batch, W3.shape[1]), jnp.float32),
    )(e, W1, b1, W2, b2, W3, b3)


def kernel(movie_ids, table, W1, b1, W2, b2, W3, b3):
    batch = movie_ids.shape[0]
    gather = _make_gather(batch, table.shape[1])
    e = gather(movie_ids.astype(jnp.int32), table)
    return _mlp(
        e,
        W1,
        b1.reshape(1, -1),
        W2,
        b2.reshape(1, -1),
        W3,
        b3.reshape(1, -1),
    )
